# baseline (speedup 1.0000x reference)
import functools

import jax
import jax.numpy as jnp
from jax import lax
from jax.experimental import pallas as pl
from jax.experimental.pallas import tpu as pltpu
from jax.experimental.pallas import tpu_sc as plsc

_NODE = 1000000
_EMB = 64
_BATCH = 16384
_G = 16
_RING = 8
_LANES = 16
_BW = 128


@jax.jit
def _lookup(inputs, table_t):
    info = plsc.get_sparse_core_info()
    num_cores, num_subcores = info.num_cores, info.num_subcores
    num_workers = num_cores * num_subcores
    bpw = _BATCH // num_workers
    n_groups = bpw // _G

    mesh = plsc.VectorSubcoreMesh(core_axis_name="c", subcore_axis_name="s")

    @functools.partial(
        pl.kernel,
        mesh=mesh,
        out_type=jax.ShapeDtypeStruct((_BATCH, _EMB), jnp.float32),
        scratch_types=[
            pltpu.VMEM((bpw,), jnp.int32),
            pltpu.VMEM((_RING, _EMB, _BW), jnp.float32),
            pltpu.VMEM((bpw // 2, _EMB), jnp.float32),
            pltpu.SemaphoreType.DMA((_RING,)),
            pltpu.SemaphoreType.DMA,
        ],
        compiler_params=pltpu.CompilerParams(needs_layout_passes=False),
    )
    def k(idx_hbm, table_hbm, out_hbm, idx_v, ring_v, rows_v, sems, isem):
        wid = lax.axis_index("s") * num_cores + lax.axis_index("c")
        base = wid * bpw
        pltpu.make_async_copy(
            idx_hbm.at[pl.ds(base, bpw)], idx_v, isem
        ).start()
        pltpu.make_async_copy(
            idx_hbm.at[pl.ds(base, bpw)], idx_v, isem
        ).wait()

        f_vecs = [lax.iota(jnp.int32, _LANES) + q * _LANES
                  for q in range(_EMB // _LANES)]

        def fetch(n, slot):
            c0 = pl.multiple_of((n // _BW) * _BW, _BW)
            pltpu.make_async_copy(
                table_hbm.at[:, pl.ds(c0, _BW)],
                ring_v.at[slot],
                sems.at[slot],
            ).start()

        def drain(slot):
            pltpu.make_async_copy(
                table_hbm.at[:, pl.ds(0, _BW)],
                ring_v.at[slot],
                sems.at[slot],
            ).wait()

        def extract(n, j, slot):
            lane_vec = jnp.full((_LANES,), n & (_BW - 1), jnp.int32)
            for q, f_vec in enumerate(f_vecs):
                v = plsc.load_gather(ring_v.at[slot], [f_vec, lane_vec])
                rows_v[j & (bpw // 2 - 1), pl.ds(q * _LANES, _LANES)] = v

        vec0 = idx_v[pl.ds(0, _G)]
        for i in range(_RING):
            fetch(vec0[i], i)

        def body(g, carry):
            jbase = g * _G
            vec = idx_v[pl.ds(jbase, _G)]
            for i in range(_RING):
                drain(i)
                extract(vec[i], jbase + i, i)
                fetch(vec[_RING + i], i)
            for i in range(_RING):
                drain(i)
                extract(vec[_RING + i], jbase + _RING + i, i)

            @pl.when(g + 1 < n_groups)
            def _():
                nvec = idx_v[pl.ds(jbase + _G, _G)]
                for i in range(_RING):
                    fetch(nvec[i], i)

            @pl.when(g == n_groups // 2 - 1)
            def _():
                pltpu.sync_copy(rows_v, out_hbm.at[pl.ds(base, bpw // 2)])

            return carry

        lax.fori_loop(0, n_groups, body, 0)
        pltpu.sync_copy(rows_v, out_hbm.at[pl.ds(base + bpw // 2, bpw // 2)])

    return k(inputs, table_t)


def kernel(inputs, embedding):
    return _lookup(inputs.astype(jnp.int32), embedding.T)


# speedup vs baseline: 1.0190x; 1.0190x over previous
import functools

import jax
import jax.numpy as jnp
from jax import lax
from jax.experimental import pallas as pl
from jax.experimental.pallas import tpu as pltpu
from jax.experimental.pallas import tpu_sc as plsc

_NODE = 1000000
_EMB = 64
_BATCH = 16384
_G = 16
_RING = 8
_LANES = 16
_BW = 128


@jax.jit
def _lookup(inputs, table_t):
    info = plsc.get_sparse_core_info()
    num_cores, num_subcores = info.num_cores, info.num_subcores
    num_workers = num_cores * num_subcores
    bpw = _BATCH // num_workers
    n_groups = bpw // _G

    mesh = plsc.VectorSubcoreMesh(core_axis_name="c", subcore_axis_name="s")

    @functools.partial(
        pl.kernel,
        mesh=mesh,
        out_type=jax.ShapeDtypeStruct((_EMB, _BATCH), jnp.float32),
        scratch_types=[
            pltpu.VMEM((bpw,), jnp.int32),
            pltpu.VMEM((_RING, _EMB, _BW), jnp.float32),
            pltpu.VMEM((_EMB, bpw // 2), jnp.float32),
            pltpu.SemaphoreType.DMA((_RING,)),
            pltpu.SemaphoreType.DMA,
        ],
        compiler_params=pltpu.CompilerParams(needs_layout_passes=False),
    )
    def k(idx_hbm, table_hbm, out_hbm, idx_v, ring_v, rows_v, sems, isem):
        wid = lax.axis_index("s") * num_cores + lax.axis_index("c")
        base = wid * bpw
        pltpu.make_async_copy(
            idx_hbm.at[pl.ds(base, bpw)], idx_v, isem
        ).start()
        pltpu.make_async_copy(
            idx_hbm.at[pl.ds(base, bpw)], idx_v, isem
        ).wait()

        f_vecs = [lax.iota(jnp.int32, _LANES) + q * _LANES
                  for q in range(_EMB // _LANES)]

        def fetch(n, slot):
            c0 = pl.multiple_of((n // _BW) * _BW, _BW)
            pltpu.make_async_copy(
                table_hbm.at[:, pl.ds(c0, _BW)],
                ring_v.at[slot],
                sems.at[slot],
            ).start()

        def drain(slot):
            pltpu.make_async_copy(
                table_hbm.at[:, pl.ds(0, _BW)],
                ring_v.at[slot],
                sems.at[slot],
            ).wait()

        def extract(n, j, slot):
            lane_vec = jnp.full((_LANES,), n & (_BW - 1), jnp.int32)
            col_vec = jnp.full((_LANES,), j & (bpw // 2 - 1), jnp.int32)
            for q, f_vec in enumerate(f_vecs):
                v = plsc.load_gather(ring_v.at[slot], [f_vec, lane_vec])
                plsc.store_scatter(rows_v, [f_vec, col_vec], v)

        vec0 = idx_v[pl.ds(0, _G)]
        for i in range(_RING):
            fetch(vec0[i], i)

        def body(g, carry):
            jbase = g * _G
            vec = idx_v[pl.ds(jbase, _G)]
            for i in range(_RING):
                drain(i)
                extract(vec[i], jbase + i, i)
                fetch(vec[_RING + i], i)
            for i in range(_RING):
                drain(i)
                extract(vec[_RING + i], jbase + _RING + i, i)

            @pl.when(g + 1 < n_groups)
            def _():
                nvec = idx_v[pl.ds(jbase + _G, _G)]
                for i in range(_RING):
                    fetch(nvec[i], i)

            @pl.when(g == n_groups // 2 - 1)
            def _():
                pltpu.sync_copy(rows_v, out_hbm.at[:, pl.ds(base, bpw // 2)])

            return carry

        lax.fori_loop(0, n_groups, body, 0)
        pltpu.sync_copy(
            rows_v, out_hbm.at[:, pl.ds(base + bpw // 2, bpw // 2)])

    return k(inputs, table_t).T


def kernel(inputs, embedding):
    return _lookup(inputs.astype(jnp.int32), embedding.T)
